# Initial kernel scaffold; baseline (speedup 1.0000x reference)
import jax
import jax.numpy as jnp
from jax.experimental import pallas as pl


def kernel(x, edge_index, edge_type, W, W_loop, bias):
    raise NotImplementedError("write your pallas kernel here")


# speedup vs baseline: 87.6138x; 87.6138x over previous
import functools

import jax
import jax.numpy as jnp
from jax import lax
from jax.experimental import pallas as pl
from jax.experimental.pallas import tpu as pltpu
from jax.experimental.pallas import tpu_sc as plsc

_NC = 2
_NS = 16
_NW = _NC * _NS
_L = 16


@functools.lru_cache(maxsize=None)
def _make_hist_kernel(E, TBL, NPAD):
    epw = E // _NW
    mesh = plsc.VectorSubcoreMesh(core_axis_name="c", subcore_axis_name="s")

    @functools.partial(
        pl.kernel,
        out_type=jax.ShapeDtypeStruct((_NW, TBL), jnp.float32),
        mesh=mesh,
        compiler_params=pltpu.CompilerParams(needs_layout_passes=False),
        scratch_types=[
            pltpu.VMEM((epw,), jnp.int32),
            pltpu.VMEM((epw,), jnp.int32),
            pltpu.VMEM((TBL,), jnp.float32),
        ],
    )
    def hist(src_hbm, typ_hbm, out_hbm, src_v, typ_v, tbl_v):
        cid = lax.axis_index("c")
        sid = lax.axis_index("s")
        wid = sid * _NC + cid
        base = wid * epw
        pltpu.sync_copy(src_hbm.at[pl.ds(base, epw)], src_v)
        pltpu.sync_copy(typ_hbm.at[pl.ds(base, epw)], typ_v)

        def zero_body(i, carry):
            tbl_v[pl.ds(i * _L, _L)] = jnp.zeros((_L,), jnp.float32)
            return carry

        lax.fori_loop(0, TBL // _L, zero_body, 0)

        ones = jnp.ones((_L,), jnp.float32)

        def edge_body(i, carry):
            s = src_v[pl.ds(i * _L, _L)]
            t = typ_v[pl.ds(i * _L, _L)]
            idx = t * NPAD + s
            plsc.addupdate_scatter(tbl_v, [idx], ones)
            return carry

        lax.fori_loop(0, epw // _L, edge_body, 0)
        pltpu.sync_copy(tbl_v, out_hbm.at[wid])

    return hist


@functools.lru_cache(maxsize=None)
def _make_combine_kernel(N, NPAD, D, R):
    RP = 8

    def body(p_ref, x_ref, w_ref, b_ref, o_ref):
        csum = jnp.sum(p_ref[...], axis=0)
        cc = jnp.concatenate(
            [csum, jnp.zeros((RP - R, NPAD), jnp.float32)], axis=0
        )
        s = jnp.dot(cc, x_ref[...], precision=lax.Precision.HIGHEST)
        xsum = jnp.sum(x_ref[...], axis=0, keepdims=True)
        row = lax.broadcasted_iota(jnp.int32, (RP, D), 0)
        s = s + jnp.where(row == R, xsum, 0.0)
        g = jnp.zeros((RP, D), jnp.float32)
        for k in range(R + 1):
            sk = jnp.where(row == k, s, 0.0)
            g = g + jnp.dot(sk, w_ref[k], precision=lax.Precision.HIGHEST)
        out = jnp.sum(g, axis=0, keepdims=True) * (1.0 / N) + b_ref[...]
        o_ref[...] = out

    return pl.pallas_call(
        body,
        out_shape=jax.ShapeDtypeStruct((1, D), jnp.float32),
    )


def kernel(x, edge_index, edge_type, W, W_loop, bias):
    N, D = x.shape
    R = W.shape[0]
    E = edge_type.shape[0]
    NPAD = ((N + 127) // 128) * 128
    TBL = R * NPAD

    src = edge_index[0]
    hist = _make_hist_kernel(E, TBL, NPAD)
    partials = hist(src, edge_type)
    partials = partials.reshape(_NW, R, NPAD)

    xp = jnp.zeros((NPAD, D), jnp.float32).at[:N].set(x)
    w5 = jnp.concatenate([W, W_loop[None]], axis=0)
    combine = _make_combine_kernel(N, NPAD, D, R)
    return combine(partials, xp, w5, bias[None])


# speedup vs baseline: 103.0019x; 1.1756x over previous
import functools

import jax
import jax.numpy as jnp
from jax import lax
from jax.experimental import pallas as pl
from jax.experimental.pallas import tpu as pltpu
from jax.experimental.pallas import tpu_sc as plsc

_NC = 2
_NS = 16
_NW = _NC * _NS
_L = 16


@functools.lru_cache(maxsize=None)
def _make_hist_kernel(E, TBL, NPAD):
    epw = E // _NW
    mesh = plsc.VectorSubcoreMesh(core_axis_name="c", subcore_axis_name="s")

    @functools.partial(
        pl.kernel,
        out_type=jax.ShapeDtypeStruct((_NW, TBL), jnp.float32),
        mesh=mesh,
        compiler_params=pltpu.CompilerParams(needs_layout_passes=False),
        scratch_types=[
            pltpu.VMEM((epw,), jnp.int32),
            pltpu.VMEM((epw,), jnp.int32),
            pltpu.VMEM((TBL,), jnp.float32),
        ],
    )
    def hist(src_hbm, typ_hbm, out_hbm, src_v, typ_v, tbl_v):
        cid = lax.axis_index("c")
        sid = lax.axis_index("s")
        wid = sid * _NC + cid
        base = wid * epw
        pltpu.sync_copy(src_hbm.at[pl.ds(base, epw)], src_v)
        pltpu.sync_copy(typ_hbm.at[pl.ds(base, epw)], typ_v)

        zero = jnp.zeros((_L,), jnp.float32)
        ZU = 16

        def zero_body(i, carry):
            for j in range(ZU):
                tbl_v[pl.ds((i * ZU + j) * _L, _L)] = zero
            return carry

        lax.fori_loop(0, TBL // (_L * ZU), zero_body, 0)

        ones = jnp.ones((_L,), jnp.float32)
        EU = 5

        def edge_body(i, carry):
            for j in range(EU):
                off = (i * EU + j) * _L
                s = src_v[pl.ds(off, _L)]
                t = typ_v[pl.ds(off, _L)]
                plsc.addupdate_scatter(tbl_v, [t * NPAD + s], ones)
            return carry

        lax.fori_loop(0, epw // (_L * EU), edge_body, 0)
        pltpu.sync_copy(tbl_v, out_hbm.at[wid])

    return hist


@functools.lru_cache(maxsize=None)
def _make_combine_kernel(N, NPAD, D, R):
    RP = 8

    def body(p_ref, x_ref, w_ref, wl_ref, b_ref, o_ref):
        csum = jnp.sum(p_ref[...], axis=0)[:, :N]
        cc = jnp.concatenate(
            [csum, jnp.zeros((RP - R, N), jnp.float32)], axis=0
        )
        s = jnp.dot(cc, x_ref[...], precision=lax.Precision.HIGHEST)
        xsum = jnp.sum(x_ref[...], axis=0, keepdims=True)
        row = lax.broadcasted_iota(jnp.int32, (RP, D), 0)
        s = s + jnp.where(row == R, xsum, 0.0)
        g = jnp.zeros((RP, D), jnp.float32)
        for k in range(R):
            sk = jnp.where(row == k, s, 0.0)
            g = g + jnp.dot(sk, w_ref[k], precision=lax.Precision.HIGHEST)
        sl = jnp.where(row == R, s, 0.0)
        g = g + jnp.dot(sl, wl_ref[...], precision=lax.Precision.HIGHEST)
        out = jnp.sum(g, axis=0, keepdims=True) * (1.0 / N) + b_ref[...]
        o_ref[...] = out

    return pl.pallas_call(
        body,
        out_shape=jax.ShapeDtypeStruct((1, D), jnp.float32),
    )


def kernel(x, edge_index, edge_type, W, W_loop, bias):
    N, D = x.shape
    R = W.shape[0]
    E = edge_type.shape[0]
    NPAD = ((N + 127) // 128) * 128
    TBL = R * NPAD

    src = edge_index[0]
    hist = _make_hist_kernel(E, TBL, NPAD)
    partials = hist(src, edge_type)
    partials = partials.reshape(_NW, R, NPAD)

    combine = _make_combine_kernel(N, NPAD, D, R)
    return combine(partials, x, W, W_loop, bias[None])
